# speedup vs baseline: 1.6453x; 1.6453x over previous
import functools

import jax
import jax.numpy as jnp
from jax import lax
from jax.experimental import pallas as pl
from jax.experimental.pallas import tpu as pltpu
from jax.experimental.pallas import tpu_sc as plsc

VOCAB = 100000
HID = 1024
BATCH = 4
SEQ = 4096

NC, NS = 2, 16
NW = NC * NS
B = BATCH * SEQ
B_PER_W = B // NW
C = 32
NCHUNK = B_PER_W // C
NBUF = 3


def _make_kernel():
    mesh = plsc.VectorSubcoreMesh(core_axis_name="c", subcore_axis_name="s")

    @functools.partial(
        pl.kernel,
        mesh=mesh,
        out_type=jax.ShapeDtypeStruct((B, HID), jnp.float32),
        scratch_types=[
            pltpu.VMEM((NCHUNK, C), jnp.int32),
            pltpu.VMEM((NBUF, C, HID), jnp.float32),
        ] + [pltpu.SemaphoreType.DMA] * (2 * NBUF),
    )
    def k(table_hbm, idx_hbm, out_hbm, idx_v, rows_v, *sems):
        gsem = sems[:NBUF]
        ssem = sems[NBUF:]
        wid = lax.axis_index("s") * NC + lax.axis_index("c")
        base = wid * B_PER_W
        pltpu.sync_copy(idx_hbm.at[wid], idx_v)

        g = {}
        s = {}
        for i in range(NBUF):
            g[i] = pltpu.async_copy(
                table_hbm.at[idx_v.at[i]], rows_v.at[i], gsem[i])
        for i in range(NCHUNK):
            b = i % NBUF
            g[i].wait()
            s[i] = pltpu.async_copy(
                rows_v.at[b], out_hbm.at[pl.ds(base + i * C, C)], ssem[b])
            j = i + NBUF
            if j < NCHUNK:
                s[i].wait()
                g[j] = pltpu.async_copy(
                    table_hbm.at[idx_v.at[j]], rows_v.at[b], gsem[b])
        for i in range(max(0, NCHUNK - NBUF), NCHUNK):
            s[i].wait()

    return k


_sc_gather = _make_kernel()


def kernel(token_ids, W):
    idx3 = token_ids.reshape(NW, NCHUNK, C)
    out = _sc_gather(W, idx3)
    return out.reshape(BATCH, SEQ, HID)
